# baseline (speedup 1.0000x reference)
import functools

import jax
import jax.numpy as jnp
from jax import lax
from jax.experimental import pallas as pl
from jax.experimental.pallas import tpu as pltpu
from jax.experimental.pallas import tpu_sc as plsc

D = 64
H = 8
OFF = 100000
NW = 32
CH = 128
NCH = 32
NBUF = 4
L = 16


def _body(ids_hbm, table_hbm, out_hbm, idx_v, rows_v, gsem):
    wid = lax.axis_index("s") * 2 + lax.axis_index("c")
    pltpu.sync_copy(ids_hbm.at[wid], idx_v)

    offs = lax.rem(lax.iota(jnp.int32, L), H) * OFF

    def shift(j, _):
        for k in range(CH // L):
            sl = pl.ds(k * L, L)
            idx_v[j, sl] = idx_v[j, sl] + offs
        return 0

    lax.fori_loop(0, NCH, shift, 0)

    def group(gi, _):
        cps = []
        for b in range(NBUF):
            j = gi * NBUF + b
            cps.append(
                pltpu.async_copy(table_hbm.at[idx_v.at[j]], rows_v.at[b],
                                 gsem.at[b]))
        for b in range(NBUF):
            cps[b].wait()
            pltpu.sync_copy(rows_v.at[b], out_hbm.at[wid, gi * NBUF + b])
        return 0

    lax.fori_loop(0, NCH // NBUF, group, 0)


def kernel(input_ids, vocab_table):
    ids = input_ids.reshape(NW, NCH, CH)
    mesh = plsc.VectorSubcoreMesh(core_axis_name="c", subcore_axis_name="s")
    out = pl.kernel(
        _body,
        out_type=jax.ShapeDtypeStruct((NW, NCH, CH, D), jnp.float32),
        mesh=mesh,
        scratch_types=[
            pltpu.VMEM((NCH, CH), jnp.int32),
            pltpu.VMEM((NBUF, CH, D), jnp.float32),
            pltpu.SemaphoreType.DMA((NBUF,)),
        ],
        compiler_params=pltpu.CompilerParams(use_tc_tiling_on_sc=False),
    )(ids, vocab_table)
    return out.reshape(input_ids.shape + (D,))


# speedup vs baseline: 1.0012x; 1.0012x over previous
import functools

import jax
import jax.numpy as jnp
from jax import lax
from jax.experimental import pallas as pl
from jax.experimental.pallas import tpu as pltpu
from jax.experimental.pallas import tpu_sc as plsc

D = 64
H = 8
OFF = 100000
NW = 32
CH = 128
NCH = 32
NBUF = 8
L = 16


def _body(ids_hbm, table_hbm, out_hbm, idx_v, rows_v, gsem, osem):
    wid = lax.axis_index("s") * 2 + lax.axis_index("c")
    pltpu.sync_copy(ids_hbm.at[wid], idx_v)

    offs = lax.rem(lax.iota(jnp.int32, L), H) * OFF

    def shift(j, _):
        for k in range(CH // L):
            sl = pl.ds(k * L, L)
            idx_v[j, sl] = idx_v[j, sl] + offs
        return 0

    lax.fori_loop(0, NCH, shift, 0)

    def g_copy(j, b):
        return pltpu.make_async_copy(table_hbm.at[idx_v.at[j]],
                                     rows_v.at[b], gsem.at[b])

    def o_copy(j, b):
        return pltpu.make_async_copy(rows_v.at[b], out_hbm.at[wid, j],
                                     osem.at[b])

    for b in range(NBUF):
        g_copy(b, b).start()

    def main(j, _):
        b = lax.rem(j, NBUF)
        g_copy(j, b).wait()
        o_copy(j, b).start()
        o_copy(j, b).wait()
        g_copy(j + NBUF, b).start()
        return 0

    lax.fori_loop(0, NCH - NBUF, main, 0)

    def epi(j, _):
        b = lax.rem(j, NBUF)
        g_copy(j, b).wait()
        o_copy(j, b).start()
        o_copy(j, b).wait()
        return 0

    lax.fori_loop(NCH - NBUF, NCH, epi, 0)


def kernel(input_ids, vocab_table):
    ids = input_ids.reshape(NW, NCH, CH)
    mesh = plsc.VectorSubcoreMesh(core_axis_name="c", subcore_axis_name="s")
    out = pl.kernel(
        _body,
        out_type=jax.ShapeDtypeStruct((NW, NCH, CH, D), jnp.float32),
        mesh=mesh,
        scratch_types=[
            pltpu.VMEM((NCH, CH), jnp.int32),
            pltpu.VMEM((NBUF, CH, D), jnp.float32),
            pltpu.SemaphoreType.DMA((NBUF,)),
            pltpu.SemaphoreType.DMA((NBUF,)),
        ],
        compiler_params=pltpu.CompilerParams(use_tc_tiling_on_sc=False),
    )(ids, vocab_table)
    return out.reshape(input_ids.shape + (D,))
